# baseline (speedup 1.0000x reference)
import jax
import jax.numpy as jnp
from jax import lax
from jax.experimental import pallas as pl
from jax.experimental.pallas import tpu as pltpu
from jax.experimental.pallas import tpu_sc as plsc

N_NODES = 10000
N_EDGES = 320000
D = 128
HID = 128
ROW = 2 * D
ROWW = ROW // 2

NC, NS, NL = 2, 16, 16
NW = NC * NS
EPW = N_EDGES // NW
K = 80
STEPS = EPW // K
NBUF = 3
UNROLL = 2


def _tab_body(hs_ref, hd_ref, rel_ref, w1_ref, b1_ref, stab_ref, dtab_ref):
    w1a = w1_ref[0:D, :]
    w1b = w1_ref[D:2 * D, :]
    w1c = w1_ref[2 * D:3 * D, :]
    rel = rel_ref[:]
    c = jnp.dot(rel, w1c, preferred_element_type=jnp.float32) + b1_ref[:]
    hs = hs_ref[:]
    hd = hd_ref[:]
    stab_ref[:, 0:D] = jnp.dot(hs, w1a, preferred_element_type=jnp.float32) + c
    stab_ref[:, D:ROW] = hs * rel
    dtab_ref[:, 0:D] = jnp.dot(hd, w1b, preferred_element_type=jnp.float32)
    dtab_ref[:, D:ROW] = hd


def _build_tables(h_src, h_dst, rel2d, W1, b1_2d):
    blk = 1000
    grid = (N_NODES // blk,)
    return pl.pallas_call(
        _tab_body,
        grid=grid,
        in_specs=[
            pl.BlockSpec((blk, D), lambda i: (i, 0)),
            pl.BlockSpec((blk, D), lambda i: (i, 0)),
            pl.BlockSpec((1, D), lambda i: (0, 0)),
            pl.BlockSpec((3 * D, D), lambda i: (0, 0)),
            pl.BlockSpec((1, D), lambda i: (0, 0)),
        ],
        out_specs=[
            pl.BlockSpec((blk, ROW), lambda i: (i, 0)),
            pl.BlockSpec((blk, ROW), lambda i: (i, 0)),
        ],
        out_shape=[
            jax.ShapeDtypeStruct((N_NODES, ROW), jnp.float32),
            jax.ShapeDtypeStruct((N_NODES, ROW), jnp.float32),
        ],
    )(h_src, h_dst, rel2d, W1, b1_2d)


def _pack_bf16(tab):
    t16 = tab.astype(jnp.bfloat16).reshape(N_NODES, ROWW, 2)
    return jax.lax.bitcast_convert_type(t16, jnp.int32)


def _lo(v):
    return lax.bitcast_convert_type(v * jnp.int32(65536), jnp.float32)


def _hi(v):
    return lax.bitcast_convert_type(v, jnp.float32)


def _edge_body(src_idx_hbm, dst_idx_hbm, stab_hbm, dtab_hbm, w2_hbm, b2_hbm,
               out_hbm, sidx, didx, sbuf, dbuf, w2v, b2v, obuf,
               semp0, semp1, semp2, osem0, osem1, osem2):
    wid = lax.axis_index("s") * NC + lax.axis_index("c")
    base = wid * EPW
    semp = (semp0, semp1, semp2)
    osems = (osem0, osem1, osem2)

    pltpu.sync_copy(w2_hbm, w2v)
    pltpu.sync_copy(b2_hbm, b2v)
    pltpu.sync_copy(src_idx_hbm.at[pl.ds(base, EPW)], sidx)
    pltpu.sync_copy(dst_idx_hbm.at[pl.ds(base, EPW)], didx)

    w2r = [w2v[pl.ds(NL * k, NL)] for k in range(HID // NL)]
    b2r = b2v[...]
    iota = lax.iota(jnp.int32, NL)
    zero = jnp.zeros((NL,), jnp.float32)
    lo8 = iota < 8
    ix8, ix4, ix2, ix1 = iota ^ 8, iota ^ 4, iota ^ 2, iota ^ 1
    ior8, iand7 = iota | 8, iota & 7

    def plain_issue(s, b):
        i = pl.ds(s * K, K)
        pltpu.async_copy(stab_hbm.at[sidx.at[i]], sbuf.at[b], semp[b])
        pltpu.async_copy(dtab_hbm.at[didx.at[i]], dbuf.at[b], semp[b])

    def plain_wait(b):
        i = pl.ds(0, K)
        pltpu.make_async_copy(stab_hbm.at[sidx.at[i]], sbuf.at[b],
                              semp[b]).wait()
        pltpu.make_async_copy(dtab_hbm.at[didx.at[i]], dbuf.at[b],
                              semp[b]).wait()

    def _perm(v, idx):
        return v.at[idx].get(mode="promise_in_bounds")

    def hsum_joint(acc_g, acc_p):
        t = jnp.where(lo8, acc_g + _perm(acc_g, ix8),
                      acc_p + _perm(acc_p, ix8))
        t = t + _perm(t, ix4)
        t = t + _perm(t, ix2)
        t = t + _perm(t, ix1)
        return _perm(t, iand7), _perm(t, ior8)

    def edge_work(b, jj):
        acc_g = None
        acc_p = None
        for m in range(D // (2 * NL)):
            va = sbuf[b, jj, pl.ds(NL * m, NL)]
            vb = dbuf[b, jj, pl.ds(NL * m, NL)]
            zE = _lo(va) + _lo(vb)
            zO = _hi(va) + _hi(vb)
            lhE = jnp.maximum(zE, 0.2 * zE)
            lhO = jnp.maximum(zO, 0.2 * zO)
            gterm = lhE * w2r[2 * m] + lhO * w2r[2 * m + 1]
            acc_g = gterm if acc_g is None else acc_g + gterm
        for m in range(D // (2 * NL)):
            vu = sbuf[b, jj, pl.ds(D // 2 + NL * m, NL)]
            vv = dbuf[b, jj, pl.ds(D // 2 + NL * m, NL)]
            pterm = _lo(vu) * _lo(vv) + _hi(vu) * _hi(vv)
            acc_p = pterm if acc_p is None else acc_p + pterm
        return hsum_joint(acc_g, acc_p)

    def compute(s, b):
        eb = base + s * K

        @pl.when(s >= NBUF)
        def _():
            pltpu.make_async_copy(obuf.at[b], out_hbm.at[pl.ds(base, K)],
                                  osems[b]).wait()

        def group(g, _):
            def edge2(j2, carry):
                gvec, pvec = carry
                for u in range(UNROLL):
                    j = j2 * UNROLL + u
                    gs, ps = edge_work(b, g * NL + j)
                    m = iota == j
                    gvec = jnp.where(m, gs, gvec)
                    pvec = jnp.where(m, ps, pvec)
                return gvec, pvec

            gvec, pvec = lax.fori_loop(0, NL // UNROLL, edge2, (zero, zero))
            gate = 1.0 / (1.0 + jnp.exp(-(gvec + b2r)))
            obuf[b, pl.ds(g * NL, NL)] = gate * pvec
            return 0

        lax.fori_loop(0, K // NL, group, 0)
        pltpu.async_copy(obuf.at[b], out_hbm.at[pl.ds(eb, K)], osems[b])

    plain_issue(0, 0)
    plain_issue(1, 1)

    def triple(s3, _):
        for db in range(NBUF):
            s = NBUF * s3 + db
            b2 = (db + 2) % NBUF
            plain_issue(s + 2, b2)
            plain_wait(db)
            compute(s, db)
        return 0

    lax.fori_loop(0, (STEPS - 2) // NBUF, triple, 0)

    s = STEPS - 2
    plain_wait(0)
    compute(s, 0)
    plain_wait(1)
    compute(s + 1, 1)

    for ob in (2, 0, 1):
        pltpu.make_async_copy(obuf.at[ob], out_hbm.at[pl.ds(base, K)],
                              osems[ob]).wait()


def _edge_kernel(src_idx, dst_idx, stab, dtab, w2p, b2vec):
    mesh = plsc.VectorSubcoreMesh(core_axis_name="c", subcore_axis_name="s")
    return pl.kernel(
        _edge_body,
        out_type=jax.ShapeDtypeStruct((N_EDGES,), jnp.float32),
        mesh=mesh,
        scratch_types=[
            pltpu.VMEM((EPW,), jnp.int32),
            pltpu.VMEM((EPW,), jnp.int32),
            pltpu.VMEM((NBUF, K, ROWW), jnp.int32),
            pltpu.VMEM((NBUF, K, ROWW), jnp.int32),
            pltpu.VMEM((HID,), jnp.float32),
            pltpu.VMEM((NL,), jnp.float32),
            pltpu.VMEM((NBUF, K), jnp.float32),
        ] + [pltpu.SemaphoreType.DMA] * 6,
    )(src_idx, dst_idx, stab, dtab, w2p, b2vec)


def kernel(edge_index, h_src, h_dst, rel_weight, W1, b1, W2, b2):
    src_idx = edge_index[0].astype(jnp.int32)
    dst_idx = edge_index[1].astype(jnp.int32)
    rel2d = rel_weight.reshape(1, D)
    b1_2d = b1.reshape(1, D)
    stab, dtab = _build_tables(h_src, h_dst, rel2d, W1, b1_2d)
    stab = _pack_bf16(stab)
    dtab = _pack_bf16(dtab)
    w2p = W2.reshape(HID // 32, NL, 2).transpose(0, 2, 1).reshape(HID)
    b2vec = jnp.broadcast_to(b2.reshape(()), (NL,))
    return _edge_kernel(src_idx, dst_idx, stab, dtab, w2p, b2vec)


# speedup vs baseline: 1.2180x; 1.2180x over previous
import jax
import jax.numpy as jnp
from jax import lax
from jax.experimental import pallas as pl
from jax.experimental.pallas import tpu as pltpu
from jax.experimental.pallas import tpu_sc as plsc

N_NODES = 10000
N_EDGES = 320000
D = 128
HID = 128

NC, NS, NL = 2, 16, 16
NW = NC * NS
EPW = N_EDGES // NW
K = 80
STEPS = EPW // K
NBUF = 3
UNROLL = 2


def _tab_body(hs_ref, hd_ref, rel_ref, w1_ref, b1_ref,
              ag_ref, bg_ref, sd_ref, hdt_ref):
    w1a = w1_ref[0:D, :]
    w1b = w1_ref[D:2 * D, :]
    w1c = w1_ref[2 * D:3 * D, :]
    rel = rel_ref[:]
    c = jnp.dot(rel, w1c, preferred_element_type=jnp.float32) + b1_ref[:]
    hs = hs_ref[:]
    hd = hd_ref[:]
    ag_ref[:] = jnp.dot(hs, w1a, preferred_element_type=jnp.float32) + c
    bg_ref[:] = jnp.dot(hd, w1b, preferred_element_type=jnp.float32)
    sd_ref[:] = hs * rel
    hdt_ref[:] = hd


def _build_tables(h_src, h_dst, rel2d, W1, b1_2d):
    blk = 1000
    grid = (N_NODES // blk,)
    return pl.pallas_call(
        _tab_body,
        grid=grid,
        in_specs=[
            pl.BlockSpec((blk, D), lambda i: (i, 0)),
            pl.BlockSpec((blk, D), lambda i: (i, 0)),
            pl.BlockSpec((1, D), lambda i: (0, 0)),
            pl.BlockSpec((3 * D, D), lambda i: (0, 0)),
            pl.BlockSpec((1, D), lambda i: (0, 0)),
        ],
        out_specs=[pl.BlockSpec((blk, D), lambda i: (i, 0))] * 4,
        out_shape=[jax.ShapeDtypeStruct((N_NODES, D), jnp.float32)] * 4,
    )(h_src, h_dst, rel2d, W1, b1_2d)


def _edge_body(src_idx_hbm, dst_idx_hbm, ag_hbm, bg_hbm, sd_hbm, hd_hbm,
               w2_hbm, b2_hbm, out_hbm, sidx, didx, zbuf, ubuf, vbuf,
               w2v, b2v, obuf,
               semp0, semp1, semp2, sema0, sema1, sema2,
               osem0, osem1, osem2):
    wid = lax.axis_index("s") * NC + lax.axis_index("c")
    base = wid * EPW
    semp = (semp0, semp1, semp2)
    sema = (sema0, sema1, sema2)
    osems = (osem0, osem1, osem2)

    pltpu.sync_copy(w2_hbm, w2v)
    pltpu.sync_copy(b2_hbm, b2v)
    pltpu.sync_copy(src_idx_hbm.at[pl.ds(base, EPW)], sidx)
    pltpu.sync_copy(dst_idx_hbm.at[pl.ds(base, EPW)], didx)

    w2r = [w2v[pl.ds(NL * k, NL)] for k in range(HID // NL)]
    b2r = b2v[...]
    iota = lax.iota(jnp.int32, NL)
    zero = jnp.zeros((NL,), jnp.float32)
    lo8 = iota < 8
    ix8, ix4, ix2, ix1 = iota ^ 8, iota ^ 4, iota ^ 2, iota ^ 1
    ior8, iand7 = iota | 8, iota & 7

    def plain_issue(s, b):
        i = pl.ds(s * K, K)
        pltpu.async_copy(ag_hbm.at[sidx.at[i]], zbuf.at[b], semp[b])
        pltpu.async_copy(sd_hbm.at[sidx.at[i]], ubuf.at[b], semp[b])
        pltpu.async_copy(hd_hbm.at[didx.at[i]], vbuf.at[b], semp[b])

    def plain_wait(b):
        i = pl.ds(0, K)
        pltpu.make_async_copy(ag_hbm.at[sidx.at[i]], zbuf.at[b],
                              semp[b]).wait()
        pltpu.make_async_copy(sd_hbm.at[sidx.at[i]], ubuf.at[b],
                              semp[b]).wait()
        pltpu.make_async_copy(hd_hbm.at[didx.at[i]], vbuf.at[b],
                              semp[b]).wait()

    def add_issue(s, b):
        pltpu.async_copy(bg_hbm.at[didx.at[pl.ds(s * K, K)]], zbuf.at[b],
                         sema[b], add=True)

    def add_wait(b):
        pltpu.make_async_copy(bg_hbm.at[didx.at[pl.ds(0, K)]], zbuf.at[b],
                              sema[b]).wait()

    def _perm(v, idx):
        return v.at[idx].get(mode="promise_in_bounds")

    def hsum_joint(acc_g, acc_p):
        t = jnp.where(lo8, acc_g + _perm(acc_g, ix8),
                      acc_p + _perm(acc_p, ix8))
        t = t + _perm(t, ix4)
        t = t + _perm(t, ix2)
        t = t + _perm(t, ix1)
        return _perm(t, iand7), _perm(t, ior8)

    def edge_work(b, jj):
        acc_g = None
        acc_p = None
        for k in range(D // NL):
            z = zbuf[b, jj, pl.ds(NL * k, NL)]
            u = ubuf[b, jj, pl.ds(NL * k, NL)]
            v = vbuf[b, jj, pl.ds(NL * k, NL)]
            lh = jnp.maximum(z, 0.2 * z)
            gterm = lh * w2r[k]
            pterm = u * v
            acc_g = gterm if acc_g is None else acc_g + gterm
            acc_p = pterm if acc_p is None else acc_p + pterm
        return hsum_joint(acc_g, acc_p)

    def compute(s, b):
        eb = base + s * K

        @pl.when(s >= NBUF)
        def _():
            pltpu.make_async_copy(obuf.at[b], out_hbm.at[pl.ds(base, K)],
                                  osems[b]).wait()

        def group(g, _):
            def edge2(j2, carry):
                gvec, pvec = carry
                for u in range(UNROLL):
                    j = j2 * UNROLL + u
                    gs, ps = edge_work(b, g * NL + j)
                    m = iota == j
                    gvec = jnp.where(m, gs, gvec)
                    pvec = jnp.where(m, ps, pvec)
                return gvec, pvec

            gvec, pvec = lax.fori_loop(0, NL // UNROLL, edge2, (zero, zero))
            gate = 1.0 / (1.0 + jnp.exp(-(gvec + b2r)))
            obuf[b, pl.ds(g * NL, NL)] = gate * pvec
            return 0

        lax.fori_loop(0, K // NL, group, 0)
        pltpu.async_copy(obuf.at[b], out_hbm.at[pl.ds(eb, K)], osems[b])

    plain_issue(0, 0)
    plain_issue(1, 1)
    plain_wait(0)
    add_issue(0, 0)

    def triple(s3, _):
        for db in range(NBUF):
            s = NBUF * s3 + db
            b1 = (db + 1) % NBUF
            b2 = (db + 2) % NBUF
            plain_wait(b1)
            add_issue(s + 1, b1)
            plain_issue(s + 2, b2)
            add_wait(db)
            compute(s, db)
        return 0

    lax.fori_loop(0, (STEPS - 2) // NBUF, triple, 0)

    s = STEPS - 2
    plain_wait(1)
    add_issue(s + 1, 1)
    add_wait(0)
    compute(s, 0)
    add_wait(1)
    compute(s + 1, 1)

    for ob in (2, 0, 1):
        pltpu.make_async_copy(obuf.at[ob], out_hbm.at[pl.ds(base, K)],
                              osems[ob]).wait()


def _edge_kernel(src_idx, dst_idx, ag, bg, sd, hd, w2, b2vec):
    mesh = plsc.VectorSubcoreMesh(core_axis_name="c", subcore_axis_name="s")
    return pl.kernel(
        _edge_body,
        out_type=jax.ShapeDtypeStruct((N_EDGES,), jnp.float32),
        mesh=mesh,
        scratch_types=[
            pltpu.VMEM((EPW,), jnp.int32),
            pltpu.VMEM((EPW,), jnp.int32),
            pltpu.VMEM((NBUF, K, D), jnp.float32),
            pltpu.VMEM((NBUF, K, D), jnp.float32),
            pltpu.VMEM((NBUF, K, D), jnp.float32),
            pltpu.VMEM((HID,), jnp.float32),
            pltpu.VMEM((NL,), jnp.float32),
            pltpu.VMEM((NBUF, K), jnp.float32),
        ] + [pltpu.SemaphoreType.DMA] * 9,
    )(src_idx, dst_idx, ag, bg, sd, hd, w2, b2vec)


def kernel(edge_index, h_src, h_dst, rel_weight, W1, b1, W2, b2):
    src_idx = edge_index[0].astype(jnp.int32)
    dst_idx = edge_index[1].astype(jnp.int32)
    rel2d = rel_weight.reshape(1, D)
    b1_2d = b1.reshape(1, D)
    ag, bg, sd, hd = _build_tables(h_src, h_dst, rel2d, W1, b1_2d)
    w2 = W2.reshape(HID)
    b2vec = jnp.broadcast_to(b2.reshape(()), (NL,))
    return _edge_kernel(src_idx, dst_idx, ag, bg, sd, hd, w2, b2vec)
